# baseline (speedup 1.0000x reference)
import jax
import jax.numpy as jnp
from jax import lax
from jax.experimental import pallas as pl
from jax.experimental.pallas import tpu as pltpu
from jax.experimental.pallas import tpu_sc as plsc

N = 10000
D = 128
DQ = D // 8
NT = 10240
NC, NS, L = 2, 16, 16
B = 128
ROWS_PT = NT // NS
NH = 10016
RT = NH // NS
_CH = [(0, 128), (128, 128), (256, 128), (384, 128), (512, 114)]


def _sc_edge_layer(hs, asrc, adst, packed):
    nb = packed.shape[1]
    mesh = plsc.VectorSubcoreMesh(
        core_axis_name="c", subcore_axis_name="s", num_cores=NC, num_subcores=NS
    )

    def body(hs_hbm, as_hbm, ad_hbm, pk_hbm, outp, den_out,
             src2, dst2, eexp2, as_t, ad_t, rows_g, rows_s, denv, hob,
             den_acc, sem_g, sem_s, sem_d):
        h_sh = hob.at[0]
        out_acc = hob.at[1]
        c = lax.axis_index("c")
        s = lax.axis_index("s")
        row0 = s * ROWS_PT
        hrow0 = s * RT

        pltpu.sync_copy(pk_hbm.at[s], src2)
        pltpu.sync_copy(as_hbm, as_t)
        pltpu.sync_copy(ad_hbm, ad_t)

        def unpk(i, carry):
            r, g = i // (B // 16), i % (B // 16)
            pv = src2[r, pl.ds(g * 16, 16)]
            dst2[r, pl.ds(g * 16, 16)] = pv & 16383
            src2[r, pl.ds(g * 16, 16)] = lax.shift_right_logical(pv, 14)
            return carry
        lax.fori_loop(0, nb * (B // 16), unpk, 0)

        zeros16 = jnp.zeros((16,), jnp.float32)

        def zden(i, carry):
            denv[pl.ds(i * 16, 16)] = zeros16
            return carry
        lax.fori_loop(0, ROWS_PT // 16, zden, 0)
        pltpu.sync_copy(denv, den_acc.at[pl.ds(row0, ROWS_PT)])

        neg = jnp.full((16,), -3e38, jnp.float32)

        def mx(i, carry):
            a, d2 = carry
            return (jnp.maximum(a, as_t[pl.ds(i * 16, 16)]),
                    jnp.maximum(d2, ad_t[pl.ds(i * 16, 16)]))
        am, dm = lax.fori_loop(0, NT // 16, mx, (neg, neg))

        dnums = lax.GatherDimensionNumbers(
            offset_dims=(), collapsed_slice_dims=(0,), start_index_map=(0,))

        def shuf(v, perm):
            return lax.gather(v, perm[:, None], dimension_numbers=dnums,
                              slice_sizes=(1,),
                              mode=lax.GatherScatterMode.PROMISE_IN_BOUNDS)

        for sh in (8, 4, 2, 1):
            perm = jnp.arange(16, dtype=jnp.int32) ^ sh
            am = jnp.maximum(am, shuf(am, perm))
            dm = jnp.maximum(dm, shuf(dm, perm))
        cmax = jnp.maximum(am + dm, 0.0)

        for p in range(4):
            pltpu.sync_copy(hs_hbm.at[c, p, pl.ds(hrow0, RT)],
                            h_sh.at[pl.ds(hrow0, RT)])

            def zrow(i, carry):
                for ch in range(DQ // 16):
                    rows_s[0, i, ch * 16:(ch + 1) * 16] = zeros16
                return carry
            lax.fori_loop(0, B, zrow, 0)
            for off, w in _CH:
                pltpu.sync_copy(rows_s.at[0, pl.ds(0, w)],
                                out_acc.at[pl.ds(hrow0 + off, w)])

            @pl.when(s == NS - 1)
            def _():
                pltpu.sync_copy(rows_s.at[0], outp.at[c, p, pl.ds(NH, B)])
                pltpu.sync_copy(rows_s.at[0, pl.ds(0, NT - NH - B)],
                                outp.at[c, p, pl.ds(NH + B, NT - NH - B)])
            plsc.subcore_barrier()

            pltpu.async_copy(h_sh.at[src2.at[0]], rows_g.at[0], sem_g)

            def blk(nb_i, carry):
                q = lax.rem(nb_i, 2)
                pltpu.make_async_copy(h_sh.at[src2.at[nb_i]], rows_g.at[q],
                                      sem_g).wait()

                @pl.when(nb_i < nb - 1)
                def _():
                    pltpu.async_copy(h_sh.at[src2.at[nb_i + 1]],
                                     rows_g.at[1 - q], sem_g)
                if p == 0:
                    for j in range(B // 16):
                        sv = src2[nb_i, j * 16:(j + 1) * 16]
                        dv = dst2[nb_i, j * 16:(j + 1) * 16]
                        e = (plsc.load_gather(as_t, [sv])
                             + plsc.load_gather(ad_t, [dv]))
                        e = jnp.where(e >= 0.0, e, e * 0.2)
                        ee = jnp.exp(e - cmax)
                        eexp2[nb_i, pl.ds(j * 16, 16)] = ee

                @pl.when(nb_i >= 2)
                def _():
                    pltpu.make_async_copy(
                        rows_s.at[q], out_acc.at[dst2.at[nb_i - 2]],
                        sem_s).wait()

                def scale(g, carry2):
                    ev16 = eexp2[nb_i, pl.ds(g * 16, 16)]
                    for l in range(16):
                        ev = jnp.full((16,), ev16[l])
                        j2 = g * 16 + l
                        for ch in range(DQ // 16):
                            sl = pl.ds(ch * 16, 16)
                            rows_s[q, j2, sl] = rows_g[q, j2, sl] * ev
                    return carry2
                lax.fori_loop(0, B // 16, scale, 0)

                pltpu.async_copy(rows_s.at[q], out_acc.at[dst2.at[nb_i]],
                                 sem_s, add=True)
                if p == 0:
                    @pl.when(c == 0)
                    def _():
                        pltpu.async_copy(eexp2.at[nb_i],
                                         den_acc.at[dst2.at[nb_i]], sem_d,
                                         add=True)
                return carry
            lax.fori_loop(0, nb, blk, 0)

            pltpu.make_async_copy(
                rows_s.at[(nb - 2) % 2], out_acc.at[dst2.at[nb - 2]],
                sem_s).wait()
            pltpu.make_async_copy(
                rows_s.at[(nb - 1) % 2], out_acc.at[dst2.at[nb - 1]],
                sem_s).wait()
            if p == 0:
                @pl.when(c == 0)
                def _():
                    def drain(i, carry):
                        pltpu.make_async_copy(
                            eexp2.at[i], den_acc.at[dst2.at[i]], sem_d).wait()
                        return carry
                    lax.fori_loop(0, nb, drain, 0)

            plsc.subcore_barrier()

            for off, w in _CH:
                pltpu.sync_copy(out_acc.at[pl.ds(hrow0 + off, w)],
                                outp.at[c, p, pl.ds(hrow0 + off, w)])

            if p == 0:
                @pl.when(c == 0)
                def _():
                    pltpu.sync_copy(den_acc.at[pl.ds(row0, ROWS_PT)],
                                    den_out.at[pl.ds(row0, ROWS_PT)])

    f = pl.kernel(
        body,
        out_type=(jax.ShapeDtypeStruct((NC, 4, NT, DQ), jnp.float32),
                  jax.ShapeDtypeStruct((NT,), jnp.float32)),
        mesh=mesh,
        compiler_params=pltpu.CompilerParams(
            needs_layout_passes=False, use_tc_tiling_on_sc=False),
        scratch_types=[
            pltpu.VMEM((nb, B), jnp.int32),
            pltpu.VMEM((nb, B), jnp.int32),
            pltpu.VMEM((nb, B), jnp.float32),
            pltpu.VMEM((NT,), jnp.float32),
            pltpu.VMEM((NT,), jnp.float32),
            pltpu.VMEM((2, B, DQ), jnp.float32),
            pltpu.VMEM((2, B, DQ), jnp.float32),
            pltpu.VMEM((ROWS_PT,), jnp.float32),
            pltpu.VMEM_SHARED((2, NH, DQ), jnp.float32),
            pltpu.VMEM_SHARED((NT,), jnp.float32),
            pltpu.SemaphoreType.DMA,
            pltpu.SemaphoreType.DMA,
            pltpu.SemaphoreType.DMA,
        ],
    )
    return f(hs, asrc, adst, packed)


def _node_block(z, w_ref, as_ref, ad_ref, h_ref, s_ref, d_ref):
    h = jnp.dot(z, w_ref[...], precision=lax.Precision.HIGHEST)
    for cc in range(2):
        for pp in range(4):
            h_ref[cc, pp] = h[:, (4 * cc + pp) * DQ:(4 * cc + pp + 1) * DQ]
    s_ref[...] = jnp.sum(h * as_ref[...], axis=1, keepdims=True)
    d_ref[...] = jnp.sum(h * ad_ref[...], axis=1, keepdims=True)


_H_SPECS = [pl.BlockSpec((2, 4, 1024, DQ), lambda i: (0, 0, i, 0)),
            pl.BlockSpec((1024, 1), lambda i: (i, 0)),
            pl.BlockSpec((1024, 1), lambda i: (i, 0))]
_H_SHAPES = [jax.ShapeDtypeStruct((NC, 4, NT, DQ), jnp.float32),
             jax.ShapeDtypeStruct((NT, 1), jnp.float32),
             jax.ShapeDtypeStruct((NT, 1), jnp.float32)]


def _merge_quarters(op_ref):
    return jnp.concatenate(
        [op_ref[cc, pp] for cc in range(2) for pp in range(4)], axis=1)


def _tc_first(xp, W, a_s, a_d):
    gb = 1024
    grid = NT // gb

    def body(x_ref, w_ref, as_ref, ad_ref, h_ref, s_ref, d_ref):
        xb = x_ref[...]
        nrm = jnp.maximum(
            jnp.sqrt(jnp.sum(xb * xb, axis=1, keepdims=True)), 1e-15)
        z = jnp.tanh(nrm) * xb / nrm
        _node_block(z, w_ref, as_ref, ad_ref, h_ref, s_ref, d_ref)

    return pl.pallas_call(
        body,
        grid=(grid,),
        in_specs=[pl.BlockSpec((gb, D), lambda i: (i, 0)),
                  pl.BlockSpec((D, D), lambda i: (0, 0)),
                  pl.BlockSpec((1, D), lambda i: (0, 0)),
                  pl.BlockSpec((1, D), lambda i: (0, 0))],
        out_specs=_H_SPECS,
        out_shape=_H_SHAPES,
    )(xp, W, a_s, a_d)


def _tc_mid(outp, denw, b, W, a_s, a_d):
    gb = 1024
    grid = NT // gb

    def body(op_ref, dw_ref, b_ref, w_ref, as_ref, ad_ref, h_ref, s_ref, d_ref):
        num = _merge_quarters(op_ref)
        den = dw_ref[...] + 1e-16
        g = num / den + b_ref[...]
        z = jnp.tanh(g) * 2.0
        _node_block(z, w_ref, as_ref, ad_ref, h_ref, s_ref, d_ref)

    return pl.pallas_call(
        body,
        grid=(grid,),
        in_specs=[pl.BlockSpec((2, 4, gb, DQ), lambda i: (0, 0, i, 0)),
                  pl.BlockSpec((gb, 1), lambda i: (i, 0)),
                  pl.BlockSpec((1, D), lambda i: (0, 0)),
                  pl.BlockSpec((D, D), lambda i: (0, 0)),
                  pl.BlockSpec((1, D), lambda i: (0, 0)),
                  pl.BlockSpec((1, D), lambda i: (0, 0))],
        out_specs=_H_SPECS,
        out_shape=_H_SHAPES,
    )(outp, denw, b, W, a_s, a_d)


def _tc_final(outp, denw, b, n_out):
    gb = 1000
    grid = n_out // gb

    def body(op_ref, dw_ref, b_ref, o_ref):
        num = _merge_quarters(op_ref)
        den = dw_ref[...] + 1e-16
        o_ref[...] = num / den + b_ref[...]

    return pl.pallas_call(
        body,
        grid=(grid,),
        in_specs=[pl.BlockSpec((2, 4, gb, DQ), lambda i: (0, 0, i, 0)),
                  pl.BlockSpec((gb, 1), lambda i: (i, 0)),
                  pl.BlockSpec((1, D), lambda i: (0, 0))],
        out_specs=pl.BlockSpec((gb, D), lambda i: (i, 0)),
        out_shape=jax.ShapeDtypeStruct((n_out, D), jnp.float32),
    )(outp, denw, b)


def kernel(x, edge_index, W1, a_src1, a_dst1, b1,
           W2, a_src2, a_dst2, b2, W3, a_src3, a_dst3, b3):
    n, d = x.shape
    e_in = edge_index.shape[1]
    etot = e_in + n
    nb = -(-etot // (NS * B))
    epad = NS * nb * B

    xp = jnp.pad(x, ((0, NT - n), (0, 0)))
    loop = jnp.arange(n, dtype=edge_index.dtype)
    src = jnp.concatenate([edge_index[0], loop]).astype(jnp.int32)
    dst = jnp.concatenate([edge_index[1], loop]).astype(jnp.int32)
    packed = src * 16384 + dst
    sentinel = jnp.int32(n * 16384 + n)
    packed = jnp.pad(packed, (0, epad - etot),
                     constant_values=sentinel).reshape(NS, nb, B)

    a1s, a1d = a_src1.reshape(1, D), a_dst1.reshape(1, D)
    a2s, a2d = a_src2.reshape(1, D), a_dst2.reshape(1, D)
    a3s, a3d = a_src3.reshape(1, D), a_dst3.reshape(1, D)

    h, s1, d1 = _tc_first(xp, W1, a1s, a1d)
    op, dw = _sc_edge_layer(h, s1.reshape(NT), d1.reshape(NT), packed)
    h, s2, d2 = _tc_mid(op, dw.reshape(NT, 1), b1.reshape(1, D), W2, a2s, a2d)
    op, dw = _sc_edge_layer(h, s2.reshape(NT), d2.reshape(NT), packed)
    h, s3, d3 = _tc_mid(op, dw.reshape(NT, 1), b2.reshape(1, D), W3, a3s, a3d)
    op, dw = _sc_edge_layer(h, s3.reshape(NT), d3.reshape(NT), packed)
    return _tc_final(op, dw.reshape(NT, 1), b3.reshape(1, D), n)


# speedup vs baseline: 1.7103x; 1.7103x over previous
import jax
import jax.numpy as jnp
from jax import lax
from jax.experimental import pallas as pl
from jax.experimental.pallas import tpu as pltpu
from jax.experimental.pallas import tpu_sc as plsc

N = 10000
D = 128
DQ = D // 8
NT = 10240
NC, NS, L = 2, 16, 16
B = 128
ROWS_PT = NT // NS
NH = 10016
RT = NH // NS
_CH = [(0, 128), (128, 128), (256, 128), (384, 128), (512, 114)]


def _sc_edge_layer(hs, asrc, adst, packed):
    nb = packed.shape[1]
    mesh = plsc.VectorSubcoreMesh(
        core_axis_name="c", subcore_axis_name="s", num_cores=NC, num_subcores=NS
    )

    def body(hs_hbm, as_hbm, ad_hbm, pk_hbm, outp, den_out,
             src2, dst2, eexp2, as_t, ad_t, rows, denv, hob, den_acc,
             sem_g, sem_s, sem_d):
        h_sh = hob.at[0]
        out_acc = hob.at[1]
        c = lax.axis_index("c")
        s = lax.axis_index("s")
        row0 = s * ROWS_PT
        hrow0 = s * RT

        pltpu.sync_copy(pk_hbm.at[s], src2)
        pltpu.sync_copy(as_hbm, as_t)
        pltpu.sync_copy(ad_hbm, ad_t)

        def unpk(i, carry):
            r, g = i // (B // 16), i % (B // 16)
            pv = src2[r, pl.ds(g * 16, 16)]
            dst2[r, pl.ds(g * 16, 16)] = pv & 16383
            src2[r, pl.ds(g * 16, 16)] = lax.shift_right_logical(pv, 14)
            return carry
        lax.fori_loop(0, nb * (B // 16), unpk, 0)

        zeros16 = jnp.zeros((16,), jnp.float32)

        def zden(i, carry):
            denv[pl.ds(i * 16, 16)] = zeros16
            return carry
        lax.fori_loop(0, ROWS_PT // 16, zden, 0)
        pltpu.sync_copy(denv, den_acc.at[pl.ds(row0, ROWS_PT)])

        neg = jnp.full((16,), -3e38, jnp.float32)

        def mx(i, carry):
            a, d2 = carry
            return (jnp.maximum(a, as_t[pl.ds(i * 16, 16)]),
                    jnp.maximum(d2, ad_t[pl.ds(i * 16, 16)]))
        am, dm = lax.fori_loop(0, NT // 16, mx, (neg, neg))

        dnums = lax.GatherDimensionNumbers(
            offset_dims=(), collapsed_slice_dims=(0,), start_index_map=(0,))

        def shuf(v, perm):
            return lax.gather(v, perm[:, None], dimension_numbers=dnums,
                              slice_sizes=(1,),
                              mode=lax.GatherScatterMode.PROMISE_IN_BOUNDS)

        for sh in (8, 4, 2, 1):
            perm = jnp.arange(16, dtype=jnp.int32) ^ sh
            am = jnp.maximum(am, shuf(am, perm))
            dm = jnp.maximum(dm, shuf(dm, perm))
        cmax = jnp.maximum(am + dm, 0.0)

        for p in range(4):
            pltpu.sync_copy(hs_hbm.at[c, p, pl.ds(hrow0, RT)],
                            h_sh.at[pl.ds(hrow0, RT)])

            def zrow(i, carry):
                for ch in range(DQ // 16):
                    rows[0, i, ch * 16:(ch + 1) * 16] = zeros16
                return carry
            lax.fori_loop(0, B, zrow, 0)
            for off, w in _CH:
                pltpu.sync_copy(rows.at[0, pl.ds(0, w)],
                                out_acc.at[pl.ds(hrow0 + off, w)])

            @pl.when(s == NS - 1)
            def _():
                pltpu.sync_copy(rows.at[0], outp.at[c, p, pl.ds(NH, B)])
                pltpu.sync_copy(rows.at[0, pl.ds(0, NT - NH - B)],
                                outp.at[c, p, pl.ds(NH + B, NT - NH - B)])
            plsc.subcore_barrier()

            pltpu.async_copy(h_sh.at[src2.at[0]], rows.at[0], sem_g)

            def blk(nb_i, carry):
                q = lax.rem(nb_i, 2)
                pltpu.make_async_copy(h_sh.at[src2.at[nb_i]], rows.at[q],
                                      sem_g).wait()

                @pl.when(nb_i >= 1)
                def _():
                    pltpu.make_async_copy(
                        rows.at[1 - q], out_acc.at[dst2.at[nb_i - 1]],
                        sem_s).wait()

                @pl.when(nb_i < nb - 1)
                def _():
                    pltpu.async_copy(h_sh.at[src2.at[nb_i + 1]],
                                     rows.at[1 - q], sem_g)
                if p == 0:
                    for j in range(B // 16):
                        sv = src2[nb_i, j * 16:(j + 1) * 16]
                        dv = dst2[nb_i, j * 16:(j + 1) * 16]
                        e = (plsc.load_gather(as_t, [sv])
                             + plsc.load_gather(ad_t, [dv]))
                        e = jnp.where(e >= 0.0, e, e * 0.2)
                        ee = jnp.exp(e - cmax)
                        eexp2[nb_i, pl.ds(j * 16, 16)] = ee

                def scale(g, carry2):
                    ev16 = eexp2[nb_i, pl.ds(g * 16, 16)]
                    for l in range(16):
                        ev = jnp.full((16,), ev16[l])
                        j2 = g * 16 + l
                        for ch in range(DQ // 16):
                            sl = pl.ds(ch * 16, 16)
                            rows[q, j2, sl] = rows[q, j2, sl] * ev
                    return carry2
                lax.fori_loop(0, B // 16, scale, 0)

                pltpu.async_copy(rows.at[q], out_acc.at[dst2.at[nb_i]],
                                 sem_s, add=True)
                if p == 0:
                    @pl.when(c == 0)
                    def _():
                        pltpu.async_copy(eexp2.at[nb_i],
                                         den_acc.at[dst2.at[nb_i]], sem_d,
                                         add=True)
                return carry
            lax.fori_loop(0, nb, blk, 0)

            pltpu.make_async_copy(
                rows.at[(nb - 1) % 2], out_acc.at[dst2.at[nb - 1]],
                sem_s).wait()
            if p == 0:
                @pl.when(c == 0)
                def _():
                    def drain(i, carry):
                        pltpu.make_async_copy(
                            eexp2.at[i], den_acc.at[dst2.at[i]], sem_d).wait()
                        return carry
                    lax.fori_loop(0, nb, drain, 0)

            plsc.subcore_barrier()

            for off, w in _CH:
                pltpu.sync_copy(out_acc.at[pl.ds(hrow0 + off, w)],
                                outp.at[c, p, pl.ds(hrow0 + off, w)])

            if p == 0:
                @pl.when(c == 0)
                def _():
                    pltpu.sync_copy(den_acc.at[pl.ds(row0, ROWS_PT)],
                                    den_out.at[pl.ds(row0, ROWS_PT)])

    f = pl.kernel(
        body,
        out_type=(jax.ShapeDtypeStruct((NC, 4, NT, DQ), jnp.float32),
                  jax.ShapeDtypeStruct((NT,), jnp.float32)),
        mesh=mesh,
        compiler_params=pltpu.CompilerParams(
            needs_layout_passes=False, use_tc_tiling_on_sc=False),
        scratch_types=[
            pltpu.VMEM((nb, B), jnp.int32),
            pltpu.VMEM((nb, B), jnp.int32),
            pltpu.VMEM((nb, B), jnp.float32),
            pltpu.VMEM((NT,), jnp.float32),
            pltpu.VMEM((NT,), jnp.float32),
            pltpu.VMEM((2, B, DQ), jnp.float32),
            pltpu.VMEM((ROWS_PT,), jnp.float32),
            pltpu.VMEM_SHARED((2, NH, DQ), jnp.float32),
            pltpu.VMEM_SHARED((NT,), jnp.float32),
            pltpu.SemaphoreType.DMA,
            pltpu.SemaphoreType.DMA,
            pltpu.SemaphoreType.DMA,
        ],
    )
    return f(hs, asrc, adst, packed)


def _node_block(z, w_ref, as_ref, ad_ref, h_ref, s_ref, d_ref):
    h = jnp.dot(z, w_ref[...], precision=lax.Precision.HIGHEST)
    for cc in range(2):
        for pp in range(4):
            h_ref[cc, pp] = h[:, (4 * cc + pp) * DQ:(4 * cc + pp + 1) * DQ]
    s_ref[...] = jnp.sum(h * as_ref[...], axis=1, keepdims=True)
    d_ref[...] = jnp.sum(h * ad_ref[...], axis=1, keepdims=True)


_H_SPECS = [pl.BlockSpec((2, 4, 1024, DQ), lambda i: (0, 0, i, 0)),
            pl.BlockSpec((1024, 1), lambda i: (i, 0)),
            pl.BlockSpec((1024, 1), lambda i: (i, 0))]
_H_SHAPES = [jax.ShapeDtypeStruct((NC, 4, NT, DQ), jnp.float32),
             jax.ShapeDtypeStruct((NT, 1), jnp.float32),
             jax.ShapeDtypeStruct((NT, 1), jnp.float32)]


def _merge_quarters(op_ref):
    return jnp.concatenate(
        [op_ref[cc, pp] for cc in range(2) for pp in range(4)], axis=1)


def _tc_first(xp, W, a_s, a_d):
    gb = 1024
    grid = NT // gb

    def body(x_ref, w_ref, as_ref, ad_ref, h_ref, s_ref, d_ref):
        xb = x_ref[...]
        nrm = jnp.maximum(
            jnp.sqrt(jnp.sum(xb * xb, axis=1, keepdims=True)), 1e-15)
        z = jnp.tanh(nrm) * xb / nrm
        _node_block(z, w_ref, as_ref, ad_ref, h_ref, s_ref, d_ref)

    return pl.pallas_call(
        body,
        grid=(grid,),
        in_specs=[pl.BlockSpec((gb, D), lambda i: (i, 0)),
                  pl.BlockSpec((D, D), lambda i: (0, 0)),
                  pl.BlockSpec((1, D), lambda i: (0, 0)),
                  pl.BlockSpec((1, D), lambda i: (0, 0))],
        out_specs=_H_SPECS,
        out_shape=_H_SHAPES,
    )(xp, W, a_s, a_d)


def _tc_mid(outp, denw, b, W, a_s, a_d):
    gb = 1024
    grid = NT // gb

    def body(op_ref, dw_ref, b_ref, w_ref, as_ref, ad_ref, h_ref, s_ref, d_ref):
        num = _merge_quarters(op_ref)
        den = dw_ref[...] + 1e-16
        g = num / den + b_ref[...]
        z = jnp.tanh(g) * 2.0
        _node_block(z, w_ref, as_ref, ad_ref, h_ref, s_ref, d_ref)

    return pl.pallas_call(
        body,
        grid=(grid,),
        in_specs=[pl.BlockSpec((2, 4, gb, DQ), lambda i: (0, 0, i, 0)),
                  pl.BlockSpec((gb, 1), lambda i: (i, 0)),
                  pl.BlockSpec((1, D), lambda i: (0, 0)),
                  pl.BlockSpec((D, D), lambda i: (0, 0)),
                  pl.BlockSpec((1, D), lambda i: (0, 0)),
                  pl.BlockSpec((1, D), lambda i: (0, 0))],
        out_specs=_H_SPECS,
        out_shape=_H_SHAPES,
    )(outp, denw, b, W, a_s, a_d)


def _tc_final(outp, denw, b, n_out):
    gb = 1000
    grid = n_out // gb

    def body(op_ref, dw_ref, b_ref, o_ref):
        num = _merge_quarters(op_ref)
        den = dw_ref[...] + 1e-16
        o_ref[...] = num / den + b_ref[...]

    return pl.pallas_call(
        body,
        grid=(grid,),
        in_specs=[pl.BlockSpec((2, 4, gb, DQ), lambda i: (0, 0, i, 0)),
                  pl.BlockSpec((gb, 1), lambda i: (i, 0)),
                  pl.BlockSpec((1, D), lambda i: (0, 0))],
        out_specs=pl.BlockSpec((gb, D), lambda i: (i, 0)),
        out_shape=jax.ShapeDtypeStruct((n_out, D), jnp.float32),
    )(outp, denw, b)


def kernel(x, edge_index, W1, a_src1, a_dst1, b1,
           W2, a_src2, a_dst2, b2, W3, a_src3, a_dst3, b3):
    n, d = x.shape
    e_in = edge_index.shape[1]
    etot = e_in + n
    nb = -(-etot // (NS * B))
    epad = NS * nb * B

    xp = jnp.pad(x, ((0, NT - n), (0, 0)))
    loop = jnp.arange(n, dtype=edge_index.dtype)
    src = jnp.concatenate([edge_index[0], loop]).astype(jnp.int32)
    dst = jnp.concatenate([edge_index[1], loop]).astype(jnp.int32)
    packed = src * 16384 + dst
    sentinel = jnp.int32(n * 16384 + n)
    packed = jnp.pad(packed, (0, epad - etot),
                     constant_values=sentinel).reshape(NS, nb, B)

    a1s, a1d = a_src1.reshape(1, D), a_dst1.reshape(1, D)
    a2s, a2d = a_src2.reshape(1, D), a_dst2.reshape(1, D)
    a3s, a3d = a_src3.reshape(1, D), a_dst3.reshape(1, D)

    h, s1, d1 = _tc_first(xp, W1, a1s, a1d)
    op, dw = _sc_edge_layer(h, s1.reshape(NT), d1.reshape(NT), packed)
    h, s2, d2 = _tc_mid(op, dw.reshape(NT, 1), b1.reshape(1, D), W2, a2s, a2d)
    op, dw = _sc_edge_layer(h, s2.reshape(NT), d2.reshape(NT), packed)
    h, s3, d3 = _tc_mid(op, dw.reshape(NT, 1), b2.reshape(1, D), W3, a3s, a3d)
    op, dw = _sc_edge_layer(h, s3.reshape(NT), d3.reshape(NT), packed)
    return _tc_final(op, dw.reshape(NT, 1), b3.reshape(1, D), n)
